# baseline (speedup 1.0000x reference)
import jax
import jax.numpy as jnp
from jax.experimental import pallas as pl
from jax.experimental.pallas import tpu as pltpu

ROW_TILE = 8
COL_TILE = 98304


def _add_kernel(patch_ref, pos_ref, out_ref):
    out_ref[...] = patch_ref[...] + pos_ref[...]


def kernel(patch, position_embedding):
    B, P, D = patch.shape
    flat = P * D
    p2 = patch.reshape(B, flat)
    pos2 = position_embedding.reshape(1, flat)
    ncols = flat // COL_TILE
    out = pl.pallas_call(
        _add_kernel,
        grid=(ncols, B // ROW_TILE),
        in_specs=[
            pl.BlockSpec((ROW_TILE, COL_TILE), lambda c, r: (r, c)),
            pl.BlockSpec((1, COL_TILE), lambda c, r: (0, c)),
        ],
        out_specs=pl.BlockSpec((ROW_TILE, COL_TILE), lambda c, r: (r, c)),
        out_shape=jax.ShapeDtypeStruct((B, flat), patch.dtype),
        compiler_params=pltpu.CompilerParams(
            dimension_semantics=("arbitrary", "arbitrary"),
        ),
    )(p2, pos2)
    return out.reshape(B, P, D)


# speedup vs baseline: 1.9171x; 1.9171x over previous
import jax
import jax.numpy as jnp
from jax.experimental import pallas as pl
from jax.experimental.pallas import tpu as pltpu

BB = 4
NBUF = 4


def _body(patch_hbm, pos_hbm, out_hbm, pos_v, ibufs, obufs, sem_pos,
          sem_in, sem_out):
    nchunk = patch_hbm.shape[0] // BB

    def in_copy(i):
        slot = i % NBUF
        return pltpu.make_async_copy(
            patch_hbm.at[pl.ds(i * BB, BB)], ibufs.at[slot], sem_in.at[slot]
        )

    def out_copy(i):
        slot = i % NBUF
        return pltpu.make_async_copy(
            obufs.at[slot], out_hbm.at[pl.ds(i * BB, BB)], sem_out.at[slot]
        )

    pos_copy = pltpu.make_async_copy(pos_hbm, pos_v, sem_pos)
    pos_copy.start()
    for i in range(NBUF):
        in_copy(i).start()
    pos_copy.wait()

    for i in range(nchunk):
        slot = i % NBUF
        in_copy(i).wait()
        if i >= NBUF:
            out_copy(i - NBUF).wait()
        obufs[slot] = ibufs[slot] + pos_v[...]
        out_copy(i).start()
        if i + NBUF < nchunk:
            in_copy(i + NBUF).start()

    for i in range(max(0, nchunk - NBUF), nchunk):
        out_copy(i).wait()


def kernel(patch, position_embedding):
    B, P, D = patch.shape
    return pl.pallas_call(
        _body,
        in_specs=[
            pl.BlockSpec(memory_space=pl.ANY),
            pl.BlockSpec(memory_space=pl.ANY),
        ],
        out_specs=pl.BlockSpec(memory_space=pl.ANY),
        out_shape=jax.ShapeDtypeStruct((B, P, D), patch.dtype),
        scratch_shapes=[
            pltpu.VMEM((P, D), patch.dtype),
            pltpu.VMEM((NBUF, BB, P, D), patch.dtype),
            pltpu.VMEM((NBUF, BB, P, D), patch.dtype),
            pltpu.SemaphoreType.DMA,
            pltpu.SemaphoreType.DMA((NBUF,)),
            pltpu.SemaphoreType.DMA((NBUF,)),
        ],
    )(patch, position_embedding)


# speedup vs baseline: 8.7136x; 4.5451x over previous
import jax
import jax.numpy as jnp
from jax.experimental import pallas as pl
from jax.experimental.pallas import tpu as pltpu

BATCH_TILE = 4


def _add_kernel(patch_ref, pos_ref, out_ref):
    out_ref[...] = patch_ref[...] + pos_ref[...]


def kernel(patch, position_embedding):
    B, P, D = patch.shape
    pt = jnp.transpose(patch, (0, 2, 1))
    post = jnp.transpose(position_embedding, (1, 0))
    out = pl.pallas_call(
        _add_kernel,
        grid=(B // BATCH_TILE,),
        in_specs=[
            pl.BlockSpec((BATCH_TILE, D, P), lambda i: (i, 0, 0)),
            pl.BlockSpec((D, P), lambda i: (0, 0)),
        ],
        out_specs=pl.BlockSpec((BATCH_TILE, D, P), lambda i: (i, 0, 0)),
        out_shape=jax.ShapeDtypeStruct((B, D, P), patch.dtype),
        compiler_params=pltpu.CompilerParams(
            dimension_semantics=("arbitrary",),
        ),
    )(pt, post)
    return jnp.transpose(out, (0, 2, 1))


# speedup vs baseline: 9.1862x; 1.0542x over previous
import jax
import jax.numpy as jnp
from jax.experimental import pallas as pl
from jax.experimental.pallas import tpu as pltpu

BATCH_TILE = 8


def _add_kernel(patch_ref, pos_ref, out_ref):
    out_ref[...] = patch_ref[...] + pos_ref[...]


def kernel(patch, position_embedding):
    B, P, D = patch.shape
    pt = jnp.transpose(patch, (0, 2, 1))
    post = jnp.transpose(position_embedding, (1, 0))
    out = pl.pallas_call(
        _add_kernel,
        grid=(B // BATCH_TILE,),
        in_specs=[
            pl.BlockSpec((BATCH_TILE, D, P), lambda i: (i, 0, 0)),
            pl.BlockSpec((D, P), lambda i: (0, 0)),
        ],
        out_specs=pl.BlockSpec((BATCH_TILE, D, P), lambda i: (i, 0, 0)),
        out_shape=jax.ShapeDtypeStruct((B, D, P), patch.dtype),
        compiler_params=pltpu.CompilerParams(
            dimension_semantics=("arbitrary",),
        ),
    )(pt, post)
    return jnp.transpose(out, (0, 2, 1))


# speedup vs baseline: 9.5954x; 1.0445x over previous
import jax
import jax.numpy as jnp
from jax.experimental import pallas as pl
from jax.experimental.pallas import tpu as pltpu

BATCH_TILE = 16


def _add_kernel(patch_ref, pos_ref, out_ref):
    out_ref[...] = patch_ref[...] + pos_ref[...]


def kernel(patch, position_embedding):
    B, P, D = patch.shape
    pt = jnp.transpose(patch, (0, 2, 1))
    post = jnp.transpose(position_embedding, (1, 0))
    out = pl.pallas_call(
        _add_kernel,
        grid=(B // BATCH_TILE,),
        in_specs=[
            pl.BlockSpec((BATCH_TILE, D, P), lambda i: (i, 0, 0)),
            pl.BlockSpec((D, P), lambda i: (0, 0)),
        ],
        out_specs=pl.BlockSpec((BATCH_TILE, D, P), lambda i: (i, 0, 0)),
        out_shape=jax.ShapeDtypeStruct((B, D, P), patch.dtype),
        compiler_params=pltpu.CompilerParams(
            dimension_semantics=("arbitrary",),
        ),
    )(pt, post)
    return jnp.transpose(out, (0, 2, 1))
